# baseline (speedup 1.0000x reference)
import functools

import jax
import jax.numpy as jnp
from jax import lax
from jax.experimental import pallas as pl
from jax.experimental.pallas import tpu as pltpu
from jax.experimental.pallas import tpu_sc as plsc

D = 32
DW = D // 2
L = 50
NC = 2
NS = 16
NW = NC * NS

CB = 8
RPC = CB * L
NBUF = 4
SUBS = ((0, 128), (128, 128), (256, 128), (384, 16))


@functools.lru_cache(maxsize=None)
def _embed_bag_kernel(B, bpw, nch):
    mesh = plsc.VectorSubcoreMesh(core_axis_name="c", subcore_axis_name="s")
    ipw = bpw * L

    @functools.partial(
        pl.kernel,
        out_type=jax.ShapeDtypeStruct((B, D), jnp.float32),
        mesh=mesh,
        compiler_params=pltpu.CompilerParams(
            needs_layout_passes=False, use_tc_tiling_on_sc=False
        ),
        scratch_types=[
            pltpu.VMEM((ipw,), jnp.int32),
            pltpu.VMEM((NBUF, RPC, DW), jnp.int32),
            pltpu.VMEM((bpw, D), jnp.float32),
            pltpu.VMEM((bpw + 8,), jnp.float32),
            pltpu.SemaphoreType.DMA((NBUF,)),
        ],
    )
    def kern(text_hbm, weight_hbm, out_hbm, idx_v, rows_v, out_v, invc_v, sems):
        wid = lax.axis_index("s") * NC + lax.axis_index("c")
        ibase = wid * ipw

        pltpu.sync_copy(text_hbm.at[pl.ds(ibase, ipw)], idx_v)

        def fire(c, b):
            for off, sz in SUBS:
                pltpu.make_async_copy(
                    weight_hbm.at[idx_v.at[pl.ds(c * RPC + off, sz)]],
                    rows_v.at[b, pl.ds(off, sz)],
                    sems.at[b],
                ).start()

        def wait(c, b):
            for off, sz in SUBS:
                pltpu.make_async_copy(
                    weight_hbm.at[idx_v.at[pl.ds(c * RPC + off, sz)]],
                    rows_v.at[b, pl.ds(off, sz)],
                    sems.at[b],
                ).wait()

        for b in range(NBUF):
            fire(b, b)

        lane_base = lax.iota(jnp.int32, 16) * L

        def cnt_group(g, carry):
            def cnt_step(l, cnt):
                v = plsc.load_gather(idx_v, [g * (16 * L) + lane_base + l])
                return cnt + jnp.where(v != 0, 1.0, 0.0).astype(jnp.float32)

            cnt = lax.fori_loop(0, L, cnt_step, jnp.zeros((16,), jnp.float32))
            invc_v[pl.ds(g * 16, 16)] = 1.0 / jnp.maximum(cnt, 1.0)
            return carry

        lax.fori_loop(0, bpw // 16, cnt_group, None)

        himask = jnp.full((16,), -65536, jnp.int32)
        col_ev = lax.iota(jnp.int32, 16) * 2
        col_od = col_ev + 1

        def group(g, carry):
            c0 = g * NBUF
            for b in range(NBUF):
                c = c0 + b
                wait(c, b)
                cvec = invc_v[pl.ds(c * CB, 16)]
                for i in range(CB):
                    acc0 = jnp.zeros((16,), jnp.float32)
                    acc1 = jnp.zeros((16,), jnp.float32)
                    for l in range(L):
                        w = rows_v[b, i * L + l, pl.ds(0, 16)]
                        lo = lax.bitcast_convert_type(
                            lax.shift_left(w, 16), jnp.float32)
                        hi = lax.bitcast_convert_type(
                            lax.bitwise_and(w, himask), jnp.float32)
                        acc0 = acc0 + lo
                        acc1 = acc1 + hi
                    bb = c * CB + i
                    s = cvec[i]
                    row = jnp.zeros((16,), jnp.int32) + bb
                    plsc.store_scatter(out_v, [row, col_ev], acc0 * s)
                    plsc.store_scatter(out_v, [row, col_od], acc1 * s)

                @pl.when(c + NBUF < nch)
                def _():
                    fire(c + NBUF, b)
            return carry

        lax.fori_loop(0, nch // NBUF, group, None)

        pltpu.sync_copy(out_v, out_hbm.at[pl.ds(wid * bpw, bpw)])

    return kern


def kernel(text, weight):
    B = text.shape[0]
    text_flat = text.astype(jnp.int32).reshape(-1)
    V = weight.shape[0]
    wb = weight.astype(jnp.bfloat16).reshape(V, DW, 2)
    wpack = lax.bitcast_convert_type(wb, jnp.int32)
    wpack = lax.optimization_barrier(wpack)
    bpw = B // NW
    return _embed_bag_kernel(B, bpw, bpw // CB)(text_flat, wpack)


# speedup vs baseline: 1.0017x; 1.0017x over previous
import functools

import jax
import jax.numpy as jnp
from jax import lax
from jax.experimental import pallas as pl
from jax.experimental.pallas import tpu as pltpu
from jax.experimental.pallas import tpu_sc as plsc

D = 32
DW = D // 2
L = 50
NC = 2
NS = 16
NW = NC * NS

CB = 8
RPC = CB * L
NBUF = 4
SUBS = ((0, 128), (128, 128), (256, 128), (384, 16))


@functools.lru_cache(maxsize=None)
def _embed_bag_kernel(B, bpw, nch):
    mesh = plsc.VectorSubcoreMesh(core_axis_name="c", subcore_axis_name="s")
    ipw = bpw * L

    @functools.partial(
        pl.kernel,
        out_type=jax.ShapeDtypeStruct((B, D), jnp.float32),
        mesh=mesh,
        compiler_params=pltpu.CompilerParams(
            needs_layout_passes=False, use_tc_tiling_on_sc=False
        ),
        scratch_types=[
            pltpu.VMEM((ipw,), jnp.int32),
            pltpu.VMEM((NBUF, RPC, DW), jnp.int32),
            pltpu.VMEM((bpw, D), jnp.float32),
            pltpu.VMEM((bpw + 8,), jnp.float32),
            pltpu.SemaphoreType.DMA((NBUF,)),
        ],
    )
    def kern(text_hbm, weight_hbm, out_hbm, idx_v, rows_v, out_v, invc_v, sems):
        wq = weight_hbm
        wid = lax.axis_index("s") * NC + lax.axis_index("c")
        ibase = wid * ipw

        pltpu.sync_copy(text_hbm.at[pl.ds(ibase, ipw)], idx_v)

        def fire(c, b):
            for off, sz in SUBS:
                pltpu.make_async_copy(
                    wq.at[idx_v.at[pl.ds(c * RPC + off, sz)]],
                    rows_v.at[b, pl.ds(off, sz)],
                    sems.at[b],
                ).start()

        def wait(c, b):
            for off, sz in SUBS:
                pltpu.make_async_copy(
                    wq.at[idx_v.at[pl.ds(c * RPC + off, sz)]],
                    rows_v.at[b, pl.ds(off, sz)],
                    sems.at[b],
                ).wait()

        for b in range(NBUF):
            fire(b, b)

        lane_base = lax.iota(jnp.int32, 16) * L

        def cnt_group(g, carry):
            def cnt_step(l, cnt):
                v = plsc.load_gather(idx_v, [g * (16 * L) + lane_base + l])
                return cnt + jnp.where(v != 0, 1.0, 0.0).astype(jnp.float32)

            cnt = lax.fori_loop(0, L, cnt_step, jnp.zeros((16,), jnp.float32))
            invc_v[pl.ds(g * 16, 16)] = 1.0 / jnp.maximum(cnt, 1.0)
            return carry

        lax.fori_loop(0, bpw // 16, cnt_group, None)

        himask = jnp.full((16,), -65536, jnp.int32)
        col_ev = lax.iota(jnp.int32, 16) * 2
        col_od = col_ev + 1

        def group(g, carry):
            c0 = g * NBUF
            for b in range(NBUF):
                c = c0 + b
                wait(c, b)
                cvec = invc_v[pl.ds(c * CB, 16)]
                for i in range(CB):
                    acc0 = jnp.zeros((16,), jnp.float32)
                    acc1 = jnp.zeros((16,), jnp.float32)
                    for l in range(L):
                        w = rows_v[b, i * L + l, pl.ds(0, 16)]
                        lo = lax.bitcast_convert_type(
                            lax.shift_left(w, 16), jnp.float32)
                        hi = lax.bitcast_convert_type(
                            lax.bitwise_and(w, himask), jnp.float32)
                        acc0 = acc0 + lo
                        acc1 = acc1 + hi
                    bb = c * CB + i
                    s = cvec[i]
                    row = jnp.zeros((16,), jnp.int32) + bb
                    plsc.store_scatter(out_v, [row, col_ev], acc0 * s)
                    plsc.store_scatter(out_v, [row, col_od], acc1 * s)

                @pl.when(c + NBUF < nch)
                def _():
                    fire(c + NBUF, b)
            return carry

        lax.fori_loop(0, nch // NBUF, group, None)

        pltpu.sync_copy(out_v, out_hbm.at[pl.ds(wid * bpw, bpw)])

    return kern


def kernel(text, weight):
    B = text.shape[0]
    text_flat = text.astype(jnp.int32).reshape(-1)
    wpack = lax.bitcast_convert_type(
        weight.astype(jnp.bfloat16).reshape(-1, DW, 2), jnp.int32)
    bpw = B // NW
    return _embed_bag_kernel(B, bpw, bpw // CB)(text_flat, wpack)


# speedup vs baseline: 1.9095x; 1.9063x over previous
import functools

import jax
import jax.numpy as jnp
from jax import lax
from jax.experimental import pallas as pl
from jax.experimental.pallas import tpu as pltpu
from jax.experimental.pallas import tpu_sc as plsc

D = 32
L = 50
NC = 2
NS = 16
NW = NC * NS

CB = 8
RPC = CB * L
NBUF = 4
SUBS = ((0, 128), (128, 128), (256, 128), (384, 16))


@functools.lru_cache(maxsize=None)
def _embed_bag_kernel(B, bpw, nch):
    mesh = plsc.VectorSubcoreMesh(core_axis_name="c", subcore_axis_name="s")
    ipw = bpw * L

    @functools.partial(
        pl.kernel,
        out_type=jax.ShapeDtypeStruct((B, D), jnp.float32),
        mesh=mesh,
        compiler_params=pltpu.CompilerParams(
            needs_layout_passes=False, use_tc_tiling_on_sc=False
        ),
        scratch_types=[
            pltpu.VMEM((ipw,), jnp.int32),
            pltpu.VMEM((NBUF, RPC, D), jnp.float32),
            pltpu.VMEM((bpw, D), jnp.float32),
            pltpu.VMEM((bpw + 8,), jnp.float32),
            pltpu.SemaphoreType.DMA((NBUF,)),
        ],
    )
    def kern(text_hbm, weight_hbm, out_hbm, idx_v, rows_v, out_v, invc_v, sems):
        wid = lax.axis_index("s") * NC + lax.axis_index("c")
        ibase = wid * ipw

        pltpu.sync_copy(text_hbm.at[pl.ds(ibase, ipw)], idx_v)

        def fire(c, b):
            for off, sz in SUBS:
                pltpu.make_async_copy(
                    weight_hbm.at[idx_v.at[pl.ds(c * RPC + off, sz)]],
                    rows_v.at[b, pl.ds(off, sz)],
                    sems.at[b],
                ).start()

        def wait(c, b):
            for off, sz in SUBS:
                pltpu.make_async_copy(
                    weight_hbm.at[idx_v.at[pl.ds(c * RPC + off, sz)]],
                    rows_v.at[b, pl.ds(off, sz)],
                    sems.at[b],
                ).wait()

        for b in range(NBUF):
            fire(b, b)

        lane_base = lax.iota(jnp.int32, 16) * L

        def cnt_group(g, carry):
            def cnt_step(l, cnt):
                v = plsc.load_gather(idx_v, [g * (16 * L) + lane_base + l])
                return cnt + jnp.where(v != 0, 1.0, 0.0).astype(jnp.float32)

            cnt = lax.fori_loop(0, L, cnt_step, jnp.zeros((16,), jnp.float32))
            invc_v[pl.ds(g * 16, 16)] = 1.0 / jnp.maximum(cnt, 1.0)
            return carry

        lax.fori_loop(0, bpw // 16, cnt_group, None)

        def group(g, carry):
            c0 = g * NBUF
            for b in range(NBUF):
                c = c0 + b
                wait(c, b)
                cvec = invc_v[pl.ds(c * CB, 16)]
                for i in range(CB):
                    acc0 = jnp.zeros((16,), jnp.float32)
                    acc1 = jnp.zeros((16,), jnp.float32)
                    for l in range(L):
                        acc0 = acc0 + rows_v[b, i * L + l, pl.ds(0, 16)]
                        acc1 = acc1 + rows_v[b, i * L + l, pl.ds(16, 16)]
                    bb = c * CB + i
                    s = cvec[i]
                    out_v[bb, pl.ds(0, 16)] = acc0 * s
                    out_v[bb, pl.ds(16, 16)] = acc1 * s

                @pl.when(c + NBUF < nch)
                def _():
                    fire(c + NBUF, b)
            return carry

        lax.fori_loop(0, nch // NBUF, group, None)

        pltpu.sync_copy(out_v, out_hbm.at[pl.ds(wid * bpw, bpw)])

    return kern


def kernel(text, weight):
    B = text.shape[0]
    text_flat = text.astype(jnp.int32).reshape(-1)
    bpw = B // NW
    return _embed_bag_kernel(B, bpw, bpw // CB)(text_flat, weight)
